# baseline (speedup 1.0000x reference)
import functools

import jax
import jax.numpy as jnp
from jax import lax
from jax.experimental import pallas as pl
from jax.experimental.pallas import tpu as pltpu
from jax.experimental.pallas import tpu_sc as plsc

B, K, Z, H = 4, 256, 128, 128
NWORK = 32
RPW = B * K // NWORK


def _prep_kernel(z_ref, p_ref, w1_ref, b1_ref, w2_ref, b2_ref,
                 m1_ref, m2_ref, nmt_ref):
    z = z_ref[0]
    m1_ref[0] = jnp.dot(z, w1_ref[...], preferred_element_type=jnp.float32) + b1_ref[...]
    m2_ref[0] = jnp.dot(z, w2_ref[...], preferred_element_type=jnp.float32) + b2_ref[...]
    neg = jnp.float32(-jnp.inf)
    nmt_ref[0] = jnp.where(p_ref[0].T != 0, jnp.float32(0), neg)


_IT = 4
_NZC = Z // 16


def _sc_body(m2_hbm, nmt_hbm, out_hbm, m2_v, nm_v, mo_v, sem):
    c = lax.axis_index("c")
    s = lax.axis_index("s")
    wid = s * 2 + c
    b = wid // (K // RPW)
    i0 = (wid % (K // RPW)) * RPW
    pltpu.sync_copy(m2_hbm.at[b], m2_v)
    pltpu.sync_copy(nmt_hbm.at[b, pl.ds(i0, RPW)], nm_v)
    neg = jnp.float32(-jnp.inf)

    def tbody(t, carry):
        def jgbody(jg, accs, t=t):
            mv = [nm_v[t * _IT + u, pl.ds(16 * jg, 16)] for u in range(_IT)]
            for jj in range(16):
                row = [m2_v[16 * jg + jj, pl.ds(16 * zc, 16)]
                       for zc in range(_NZC)]
                accs = tuple(
                    tuple(jnp.maximum(accs[u][zc], row[zc] + mv[u][jj])
                          for zc in range(_NZC))
                    for u in range(_IT))
            return accs

        init = tuple(tuple(jnp.full((16,), neg, jnp.float32)
                           for _ in range(_NZC)) for _ in range(_IT))
        accs = lax.fori_loop(0, K // 16, jgbody, init)
        for u in range(_IT):
            for zc in range(_NZC):
                mo_v[t * _IT + u, pl.ds(16 * zc, 16)] = accs[u][zc]
        return carry

    lax.fori_loop(0, RPW // _IT, tbody, 0)
    pltpu.sync_copy(mo_v, out_hbm.at[b, pl.ds(i0, RPW)])


def _post_kernel(z_ref, m1_ref, mm_ref, wut_ref, wub_ref, bu_ref, out_ref):
    neg = jnp.float32(-jnp.inf)
    M = mm_ref[0]
    m = jnp.where(M == neg, neg, jax.nn.relu(m1_ref[0] + M))
    acc = jnp.dot(z_ref[0], wut_ref[...], preferred_element_type=jnp.float32)
    acc = acc + jnp.dot(m, wub_ref[...], preferred_element_type=jnp.float32)
    out_ref[0] = jax.nn.relu(acc + bu_ref[...])


@jax.jit
def kernel(z, P, W_M1, b_M1, W_M2, b_M2, W_U, b_U):
    m1, m2, nmt = pl.pallas_call(
        _prep_kernel,
        grid=(B,),
        in_specs=[
            pl.BlockSpec((1, K, Z), lambda b: (b, 0, 0)),
            pl.BlockSpec((1, K, K), lambda b: (b, 0, 0)),
            pl.BlockSpec((Z, Z), lambda b: (0, 0)),
            pl.BlockSpec((1, Z), lambda b: (0, 0)),
            pl.BlockSpec((Z, Z), lambda b: (0, 0)),
            pl.BlockSpec((1, Z), lambda b: (0, 0)),
        ],
        out_specs=(
            pl.BlockSpec((1, K, Z), lambda b: (b, 0, 0)),
            pl.BlockSpec((1, K, Z), lambda b: (b, 0, 0)),
            pl.BlockSpec((1, K, K), lambda b: (b, 0, 0)),
        ),
        out_shape=(
            jax.ShapeDtypeStruct((B, K, Z), jnp.float32),
            jax.ShapeDtypeStruct((B, K, Z), jnp.float32),
            jax.ShapeDtypeStruct((B, K, K), jnp.float32),
        ),
    )(z, P, W_M1, b_M1.reshape(1, Z), W_M2, b_M2.reshape(1, Z))

    mesh = plsc.VectorSubcoreMesh(core_axis_name="c", subcore_axis_name="s")
    sc_agg = functools.partial(
        pl.kernel,
        mesh=mesh,
        out_type=jax.ShapeDtypeStruct((B, K, Z), jnp.float32),
        scratch_types=[
            pltpu.VMEM((K, Z), jnp.float32),
            pltpu.VMEM((RPW, K), jnp.float32),
            pltpu.VMEM((RPW, Z), jnp.float32),
            pltpu.SemaphoreType.DMA,
        ],
    )(_sc_body)
    M = sc_agg(m2, nmt)

    out = pl.pallas_call(
        _post_kernel,
        grid=(B,),
        in_specs=[
            pl.BlockSpec((1, K, Z), lambda b: (b, 0, 0)),
            pl.BlockSpec((1, K, Z), lambda b: (b, 0, 0)),
            pl.BlockSpec((1, K, Z), lambda b: (b, 0, 0)),
            pl.BlockSpec((Z, H), lambda b: (0, 0)),
            pl.BlockSpec((Z, H), lambda b: (0, 0)),
            pl.BlockSpec((1, H), lambda b: (0, 0)),
        ],
        out_specs=pl.BlockSpec((1, K, H), lambda b: (b, 0, 0)),
        out_shape=jax.ShapeDtypeStruct((B, K, H), jnp.float32),
    )(z, m1, M, W_U[:Z], W_U[Z:], b_U.reshape(1, H))
    return out


# speedup vs baseline: 1.4320x; 1.4320x over previous
import functools

import jax
import jax.numpy as jnp
from jax import lax
from jax.experimental import pallas as pl
from jax.experimental.pallas import tpu as pltpu
from jax.experimental.pallas import tpu_sc as plsc

B, K, Z, H = 4, 256, 128, 128
ISC = 64
KTC = K - ISC
NWORK = 32
WPB = NWORK // B
RPW = ISC // WPB
_IT = 4
_NZC = Z // 16


def _prep_kernel(z_ref, p_ref, w1_ref, b1_ref, w2_ref, b2_ref,
                 m1_ref, m2_ref, nmt_ref):
    z = z_ref[0]
    m1_ref[0] = jnp.dot(z, w1_ref[...], preferred_element_type=jnp.float32) + b1_ref[...]
    m2_ref[0] = jnp.dot(z, w2_ref[...], preferred_element_type=jnp.float32) + b2_ref[...]
    neg = jnp.float32(-jnp.inf)
    nmt_ref[0] = jnp.where(p_ref[0].T != 0, jnp.float32(0), neg)


def _sc_body(m2_hbm, nmt_hbm, out_hbm, m2_v, nm_v, mo_v, sem):
    c = lax.axis_index("c")
    s = lax.axis_index("s")
    wid = s * 2 + c
    b = wid // WPB
    i0 = (wid % WPB) * RPW
    pltpu.sync_copy(m2_hbm.at[b], m2_v)
    pltpu.sync_copy(nmt_hbm.at[b, pl.ds(i0, RPW)], nm_v)
    neg = jnp.float32(-jnp.inf)

    def tbody(t, carry):
        def jgbody(jg, accs, t=t):
            mv = [nm_v[t * _IT + u, pl.ds(16 * jg, 16)] for u in range(_IT)]
            for jj in range(16):
                row = [m2_v[16 * jg + jj, pl.ds(16 * zc, 16)]
                       for zc in range(_NZC)]
                accs = tuple(
                    tuple(jnp.maximum(accs[u][zc], row[zc] + mv[u][jj])
                          for zc in range(_NZC))
                    for u in range(_IT))
            return accs

        init = tuple(tuple(jnp.full((16,), neg, jnp.float32)
                           for _ in range(_NZC)) for _ in range(_IT))
        accs = lax.fori_loop(0, K // 16, jgbody, init)
        for u in range(_IT):
            for zc in range(_NZC):
                mo_v[t * _IT + u, pl.ds(16 * zc, 16)] = accs[u][zc]
        return carry

    lax.fori_loop(0, RPW // _IT, tbody, 0)
    pltpu.sync_copy(mo_v, out_hbm.at[b, pl.ds(i0, RPW)])


def _tcmax_kernel(p_ref, m2_ref, mt_ref):
    m2 = m2_ref[0]
    neg = jnp.float32(-jnp.inf)
    nm = jnp.where(p_ref[0] != 0, jnp.float32(0), neg)
    rows = []
    for i in range(ISC, K):
        s = m2 + nm[:, i:i + 1]
        rows.append(jnp.max(s, axis=0, keepdims=True))
    mt_ref[0] = jnp.concatenate(rows, axis=0)


def _post_kernel(z_ref, m1_ref, msc_ref, mtc_ref, wut_ref, wub_ref, bu_ref,
                 out_ref):
    neg = jnp.float32(-jnp.inf)
    M = jnp.concatenate([msc_ref[0], mtc_ref[0]], axis=0)
    m = jnp.where(M == neg, neg, jax.nn.relu(m1_ref[0] + M))
    acc = jnp.dot(z_ref[0], wut_ref[...], preferred_element_type=jnp.float32)
    acc = acc + jnp.dot(m, wub_ref[...], preferred_element_type=jnp.float32)
    out_ref[0] = jax.nn.relu(acc + bu_ref[...])


@jax.jit
def kernel(z, P, W_M1, b_M1, W_M2, b_M2, W_U, b_U):
    m1, m2, nmt = pl.pallas_call(
        _prep_kernel,
        grid=(B,),
        in_specs=[
            pl.BlockSpec((1, K, Z), lambda b: (b, 0, 0)),
            pl.BlockSpec((1, K, K), lambda b: (b, 0, 0)),
            pl.BlockSpec((Z, Z), lambda b: (0, 0)),
            pl.BlockSpec((1, Z), lambda b: (0, 0)),
            pl.BlockSpec((Z, Z), lambda b: (0, 0)),
            pl.BlockSpec((1, Z), lambda b: (0, 0)),
        ],
        out_specs=(
            pl.BlockSpec((1, K, Z), lambda b: (b, 0, 0)),
            pl.BlockSpec((1, K, Z), lambda b: (b, 0, 0)),
            pl.BlockSpec((1, K, K), lambda b: (b, 0, 0)),
        ),
        out_shape=(
            jax.ShapeDtypeStruct((B, K, Z), jnp.float32),
            jax.ShapeDtypeStruct((B, K, Z), jnp.float32),
            jax.ShapeDtypeStruct((B, K, K), jnp.float32),
        ),
    )(z, P, W_M1, b_M1.reshape(1, Z), W_M2, b_M2.reshape(1, Z))

    mesh = plsc.VectorSubcoreMesh(core_axis_name="c", subcore_axis_name="s")
    sc_agg = functools.partial(
        pl.kernel,
        mesh=mesh,
        out_type=jax.ShapeDtypeStruct((B, ISC, Z), jnp.float32),
        scratch_types=[
            pltpu.VMEM((K, Z), jnp.float32),
            pltpu.VMEM((RPW, K), jnp.float32),
            pltpu.VMEM((RPW, Z), jnp.float32),
            pltpu.SemaphoreType.DMA,
        ],
    )(_sc_body)
    M_sc = sc_agg(m2, nmt)

    M_tc = pl.pallas_call(
        _tcmax_kernel,
        grid=(B,),
        in_specs=[
            pl.BlockSpec((1, K, K), lambda b: (b, 0, 0)),
            pl.BlockSpec((1, K, Z), lambda b: (b, 0, 0)),
        ],
        out_specs=pl.BlockSpec((1, KTC, Z), lambda b: (b, 0, 0)),
        out_shape=jax.ShapeDtypeStruct((B, KTC, Z), jnp.float32),
    )(P, m2)

    out = pl.pallas_call(
        _post_kernel,
        grid=(B,),
        in_specs=[
            pl.BlockSpec((1, K, Z), lambda b: (b, 0, 0)),
            pl.BlockSpec((1, K, Z), lambda b: (b, 0, 0)),
            pl.BlockSpec((1, ISC, Z), lambda b: (b, 0, 0)),
            pl.BlockSpec((1, KTC, Z), lambda b: (b, 0, 0)),
            pl.BlockSpec((Z, H), lambda b: (0, 0)),
            pl.BlockSpec((Z, H), lambda b: (0, 0)),
            pl.BlockSpec((1, H), lambda b: (0, 0)),
        ],
        out_specs=pl.BlockSpec((1, K, H), lambda b: (b, 0, 0)),
        out_shape=jax.ShapeDtypeStruct((B, K, H), jnp.float32),
    )(z, m1, M_sc, M_tc, W_U[:Z], W_U[Z:], b_U.reshape(1, H))
    return out


# speedup vs baseline: 2.5825x; 1.8034x over previous
import jax
import jax.numpy as jnp
from jax.experimental import pallas as pl

B, K, Z, H = 4, 256, 128, 128


def _fused_kernel(z_ref, p_ref, w1_ref, b1_ref, w2_ref, b2_ref,
                  wu_ref, bu_ref, out_ref):
    z = z_ref[0]
    m2 = jnp.dot(z, w2_ref[...], preferred_element_type=jnp.float32) + b2_ref[...]
    neg = jnp.float32(-jnp.inf)
    nm = jnp.where(p_ref[0] != 0, jnp.float32(0), neg)

    rows = []
    for i in range(K):
        s = m2 + nm[:, i:i + 1]
        rows.append(jnp.max(s, axis=0, keepdims=True))
    M = jnp.concatenate(rows, axis=0)

    m1 = jnp.dot(z, w1_ref[...], preferred_element_type=jnp.float32) + b1_ref[...]
    m = jnp.where(M == neg, neg, jax.nn.relu(m1 + M))
    acc = jnp.dot(z, wu_ref[:Z], preferred_element_type=jnp.float32)
    acc = acc + jnp.dot(m, wu_ref[Z:], preferred_element_type=jnp.float32)
    out_ref[0] = jax.nn.relu(acc + bu_ref[...])


@jax.jit
def kernel(z, P, W_M1, b_M1, W_M2, b_M2, W_U, b_U):
    return pl.pallas_call(
        _fused_kernel,
        grid=(B,),
        in_specs=[
            pl.BlockSpec((1, K, Z), lambda b: (b, 0, 0)),
            pl.BlockSpec((1, K, K), lambda b: (b, 0, 0)),
            pl.BlockSpec((Z, Z), lambda b: (0, 0)),
            pl.BlockSpec((1, Z), lambda b: (0, 0)),
            pl.BlockSpec((Z, Z), lambda b: (0, 0)),
            pl.BlockSpec((1, Z), lambda b: (0, 0)),
            pl.BlockSpec((2 * Z, H), lambda b: (0, 0)),
            pl.BlockSpec((1, H), lambda b: (0, 0)),
        ],
        out_specs=pl.BlockSpec((1, K, H), lambda b: (b, 0, 0)),
        out_shape=jax.ShapeDtypeStruct((B, K, H), jnp.float32),
    )(z, P, W_M1, b_M1.reshape(1, Z), W_M2, b_M2.reshape(1, Z),
      W_U, b_U.reshape(1, H))
